# baseline (speedup 1.0000x reference)
import jax
import jax.numpy as jnp
from jax import lax
from jax.experimental import pallas as pl
from jax.experimental.pallas import tpu as pltpu
from jax.experimental.pallas import tpu_sc as plsc

N = 10000
E = 640000
B = 32
D64 = 64
F = 32
NC = 2
NS = 16
NW = NC * NS
G = 125
JPC = 4
CHUNK = G * JPC
ROWS2D = E // G
TILE_ROWS = ROWS2D // NW
NCHUNK = TILE_ROWS // JPC
NPT = N // NS
EP2 = E // 2

def _sc_mesh():
  return plsc.VectorSubcoreMesh(
      core_axis_name="c", subcore_axis_name="s", num_cores=NC, num_subcores=NS)


def _sc_gather(xin64, row2d, col2d, hoff, nrows):
  tile_rows = nrows // NW
  nchunk = tile_rows // JPC
  ne = nrows * G

  def body(xin_hbm, row2d_hbm, col2d_hbm, fr_hbm, fc_hbm,
           idx_r, idx_c, buf_r, buf_c, sem):
    c = lax.axis_index("c")
    s = lax.axis_index("s")
    w = s * NC + c
    base2d = hoff + w * tile_rows

    @pl.loop(0, nchunk)
    def _chunk(k):
      rbase = base2d + k * JPC
      pltpu.sync_copy(row2d_hbm.at[pl.ds(rbase, JPC)], idx_r)
      pltpu.sync_copy(col2d_hbm.at[pl.ds(rbase, JPC)], idx_c)
      descs = []
      for j in range(JPC):
        descs.append(pltpu.async_copy(
            xin_hbm.at[idx_r.at[j]], buf_r.at[pl.ds(j * G, G)], sem))
        descs.append(pltpu.async_copy(
            xin_hbm.at[idx_c.at[j]], buf_c.at[pl.ds(j * G, G)], sem))
      for d in descs:
        d.wait()
      ebase = (rbase - hoff) * G
      pltpu.sync_copy(buf_r, fr_hbm.at[pl.ds(ebase, CHUNK)])
      pltpu.sync_copy(buf_c, fc_hbm.at[pl.ds(ebase, CHUNK)])

  return pl.kernel(
      body,
      out_type=(jax.ShapeDtypeStruct((ne, D64), jnp.float32),
                jax.ShapeDtypeStruct((ne, D64), jnp.float32)),
      mesh=_sc_mesh(),
      scratch_types=(
          pltpu.VMEM((JPC, G), jnp.int32),
          pltpu.VMEM((JPC, G), jnp.int32),
          pltpu.VMEM((CHUNK, D64), jnp.float32),
          pltpu.VMEM((CHUNK, D64), jnp.float32),
          pltpu.SemaphoreType.DMA,
      ),
      compiler_params=pltpu.CompilerParams(use_tc_tiling_on_sc=False),
  )(xin64, row2d, col2d)


def _sc_scatter(m64, row2d, zeros64, hoff, nrows):
  tile_rows = nrows // NW
  nchunk = tile_rows // JPC

  def body(m64_hbm, row2d_hbm, zeros_hbm, out_hbm, idx_r, vals, sem, acc):
    c = lax.axis_index("c")
    s = lax.axis_index("s")
    w = s * NC + c
    base2d = hoff + w * tile_rows

    pltpu.sync_copy(zeros_hbm.at[pl.ds(s * NPT, NPT)],
                    acc.at[pl.ds(s * NPT, NPT)])
    plsc.subcore_barrier()

    @pl.loop(0, nchunk)
    def _chunk(k):
      rbase = base2d + k * JPC
      pltpu.sync_copy(row2d_hbm.at[pl.ds(rbase, JPC)], idx_r)
      pltpu.sync_copy(m64_hbm.at[pl.ds((rbase - hoff) * G, CHUNK)], vals)
      descs = []
      for j in range(JPC):
        descs.append(pltpu.async_copy(
            vals.at[pl.ds(j * G, G)], acc.at[idx_r.at[j]], sem, add=True))
      for d in descs:
        d.wait()

    plsc.subcore_barrier()
    pltpu.sync_copy(acc.at[pl.ds(s * NPT, NPT)],
                    out_hbm.at[c, pl.ds(s * NPT, NPT)])

  return pl.kernel(
      body,
      out_type=jax.ShapeDtypeStruct((NC, N, D64), jnp.float32),
      mesh=_sc_mesh(),
      scratch_types=(
          pltpu.VMEM((JPC, G), jnp.int32),
          pltpu.VMEM((CHUNK, D64), jnp.float32),
          pltpu.SemaphoreType.DMA,
          pltpu.VMEM_SHARED((N, D64), jnp.float32),
      ),
      compiler_params=pltpu.CompilerParams(use_tc_tiling_on_sc=False),
  )(m64, row2d, zeros64)


_NBLK = 1000
_NGRID = N // _NBLK
_EBLK = 4000
_E2BLK = _EBLK // 2
_EGRID = E // _EBLK


def _prologue_body(x_ref, mu_ref, pos_ref, wx_ref, wmu_ref, b_ref, out_ref):
  proj = (jnp.dot(x_ref[...], wx_ref[...], preferred_element_type=jnp.float32)
          + jnp.dot(mu_ref[...], wmu_ref[...],
                    preferred_element_type=jnp.float32)
          + b_ref[...])
  pad = jnp.zeros((out_ref.shape[0], D64 - 3 - F), jnp.float32)
  out_ref[...] = jnp.concatenate([pos_ref[...], proj, pad], axis=1)


def _prologue(x, mu, pos, wx, wmu, b):
  return pl.pallas_call(
      _prologue_body,
      grid=(_NGRID,),
      in_specs=[
          pl.BlockSpec((_NBLK, 128), lambda i: (i, 0)),
          pl.BlockSpec((_NBLK, 5), lambda i: (i, 0)),
          pl.BlockSpec((_NBLK, 3), lambda i: (i, 0)),
          pl.BlockSpec((128, F), lambda i: (0, 0)),
          pl.BlockSpec((5, F), lambda i: (0, 0)),
          pl.BlockSpec((1, F), lambda i: (0, 0)),
      ],
      out_specs=pl.BlockSpec((_NBLK, D64), lambda i: (i, 0)),
      out_shape=jax.ShapeDtypeStruct((N, D64), jnp.float32),
  )(x, mu, pos, wx, wmu, b)


def _edge_mlp_body(fr2_ref, fc2_ref, ea_ref,
                   w1r_ref, w1c_ref, w1e_ref, w1d_ref, b1_ref,
                   w2_ref, b2_ref, cw1_ref, cb1_ref, cw2_ref, cb2_ref,
                   sm0_ref, sr0_ref, sm1_ref, sr1_ref, onesrow_ref,
                   out_ref):
  fr2 = fr2_ref[...]
  fc2 = fc2_ref[...]
  fr = jnp.concatenate([fr2[:, 0:D64], fr2[:, D64:2 * D64]], axis=0)
  fc = jnp.concatenate([fc2[:, 0:D64], fc2[:, D64:2 * D64]], axis=0)
  ea = ea_ref[...]
  rel = fr[:, 0:3] - fc[:, 0:3]
  rd = jnp.sum(rel * rel, axis=1, keepdims=True)
  hpre = (jnp.dot(fr[:, 3:3 + F], w1r_ref[...],
                  preferred_element_type=jnp.float32)
          + jnp.dot(fc[:, 3:3 + F], w1c_ref[...],
                    preferred_element_type=jnp.float32)
          + jnp.dot(ea, w1e_ref[...],
                    preferred_element_type=jnp.float32)
          + rd * w1d_ref[...] + b1_ref[...])
  h = jax.nn.silu(hpre)
  m = jax.nn.silu(jnp.dot(h, w2_ref[...],
                          preferred_element_type=jnp.float32) + b2_ref[...])
  h2 = jax.nn.silu(jnp.dot(m, cw1_ref[...],
                           preferred_element_type=jnp.float32) + cb1_ref[...])
  cw = jnp.dot(h2, cw2_ref[...],
               preferred_element_type=jnp.float32) + cb2_ref[...]
  rc = rel * cw
  out_ref[...] = (
      jnp.dot(m[0:_E2BLK], sm0_ref[...], preferred_element_type=jnp.float32)
      + jnp.dot(rc[0:_E2BLK], sr0_ref[...], preferred_element_type=jnp.float32)
      + jnp.dot(m[_E2BLK:], sm1_ref[...], preferred_element_type=jnp.float32)
      + jnp.dot(rc[_E2BLK:], sr1_ref[...], preferred_element_type=jnp.float32)
      + onesrow_ref[...])


def _pack_selectors():
  i32 = jnp.int32
  r = lambda n: jnp.arange(n)
  sm0 = jnp.zeros((F, 128), jnp.float32).at[r(F), r(F)].set(1.0)
  sm1 = jnp.zeros((F, 128), jnp.float32).at[r(F), D64 + r(F)].set(1.0)
  sr0 = jnp.zeros((3, 128), jnp.float32).at[r(3), F + r(3)].set(1.0)
  sr1 = jnp.zeros((3, 128), jnp.float32).at[r(3), D64 + F + r(3)].set(1.0)
  onesrow = jnp.zeros((1, 128), jnp.float32).at[
      0, jnp.array([F + 3, D64 + F + 3], i32)].set(1.0)
  return sm0, sr0, sm1, sr1, onesrow


def _edge_mlp(fr2, fc2, edge_attr, lp, hblk, sels):
  ein = 2 * F + 16 + 1
  w1 = lp["e_w1"]
  args = (fr2, fc2, edge_attr,
          w1[0:F], w1[F:2 * F], w1[2 * F:2 * F + 16], w1[2 * F + 16:ein],
          lp["e_b1"].reshape(1, -1),
          lp["e_w2"], lp["e_b2"].reshape(1, -1),
          lp["c_w1"], lp["c_b1"].reshape(1, -1),
          lp["c_w2"], lp["c_b2"].reshape(1, -1)) + sels
  full = lambda a: pl.BlockSpec(a.shape, lambda i: tuple(0 for _ in a.shape))
  return pl.pallas_call(
      _edge_mlp_body,
      grid=(fr2.shape[0] // _E2BLK,),
      in_specs=[
          pl.BlockSpec((_E2BLK, 128), lambda i: (i, 0)),
          pl.BlockSpec((_E2BLK, 128), lambda i: (i, 0)),
          pl.BlockSpec((_EBLK, 16), lambda i: (i + hblk, 0)),
      ] + [full(a) for a in args[3:]],
      out_specs=pl.BlockSpec((_E2BLK, 128), lambda i: (i, 0)),
      out_shape=jax.ShapeDtypeStruct((fr2.shape[0], 128), jnp.float32),
  )(*args)


def _node_update_body(n_acc, *refs):
  (xin_ref, *rest) = refs
  acc_refs = rest[:n_acc]
  w1f_ref, w1m_ref, b1_ref, w2_ref, b2_ref, out_ref = rest[n_acc:]
  xin = xin_ref[...]
  a = acc_refs[0][...]
  for r in acc_refs[1:]:
    a = a + r[...]
  m_i = a[:, 0:F]
  csum = a[:, F:F + 3]
  deg = a[:, F + 3:F + 4]
  coors = xin[:, 0:3]
  feats = xin[:, 3:3 + F]
  coors_out = coors + csum / jnp.maximum(deg, 1.0)
  nh = jax.nn.silu(
      jnp.dot(feats, w1f_ref[...], preferred_element_type=jnp.float32)
      + jnp.dot(m_i, w1m_ref[...], preferred_element_type=jnp.float32)
      + b1_ref[...])
  feats_out = feats + jnp.dot(nh, w2_ref[...],
                              preferred_element_type=jnp.float32) + b2_ref[...]
  pad = jnp.zeros((out_ref.shape[0], D64 - 3 - F), jnp.float32)
  out_ref[...] = jnp.concatenate(
      [coors + coors_out, feats + feats_out, pad], axis=1)


def _node_update(xin64, accs, lp):
  import functools
  w1 = lp["n_w1"]
  acc_args = tuple(a[i] for a in accs for i in range(NC))
  n_acc = len(acc_args)
  args = (xin64,) + acc_args + (w1[0:F], w1[F:2 * F],
          lp["n_b1"].reshape(1, -1), lp["n_w2"], lp["n_b2"].reshape(1, -1))
  full = lambda a: pl.BlockSpec(a.shape, lambda i: tuple(0 for _ in a.shape))
  nspec = pl.BlockSpec((_NBLK, D64), lambda i: (i, 0))
  return pl.pallas_call(
      functools.partial(_node_update_body, n_acc),
      grid=(_NGRID,),
      in_specs=[nspec] * (1 + n_acc) + [full(a) for a in args[1 + n_acc:]],
      out_specs=pl.BlockSpec((_NBLK, D64), lambda i: (i, 0)),
      out_shape=jax.ShapeDtypeStruct((N, D64), jnp.float32),
  )(*args)


def _pool_protein_body(p_ref, bf_ref, out_ref):
  i = pl.program_id(1)
  onehot = (bf_ref[...] == lax.broadcasted_iota(
      jnp.int32, (bf_ref.shape[0], B), 1).astype(jnp.float32)
            ).astype(jnp.float32)
  part = lax.dot_general(onehot, p_ref[...], (((0,), (0,)), ((), ())),
                         preferred_element_type=jnp.float32)

  @pl.when(i == 0)
  def _():
    out_ref[...] = jnp.zeros_like(out_ref)

  out_ref[...] += part


def _pool_protein(protein_x, batchf):
  FB = 512
  return pl.pallas_call(
      _pool_protein_body,
      grid=(2560 // FB, _NGRID),
      in_specs=[
          pl.BlockSpec((_NBLK, FB), lambda j, i: (i, j)),
          pl.BlockSpec((_NBLK, 1), lambda j, i: (i, 0)),
      ],
      out_specs=pl.BlockSpec((B, FB), lambda j, i: (0, j)),
      out_shape=jax.ShapeDtypeStruct((B, 2560), jnp.float32),
  )(protein_x, batchf)


def _pool_temb_body(t_ref, bf_ref, emb_ref, tsum_ref, cnt_ref):
  i = pl.program_id(0)
  rows = t_ref.shape[0]
  b1h = (bf_ref[...] == lax.broadcasted_iota(
      jnp.int32, (rows, B), 1).astype(jnp.float32)).astype(jnp.float32)
  t1h = (t_ref[...] == lax.broadcasted_iota(
      jnp.int32, (rows, 21), 1).astype(jnp.float32)).astype(jnp.float32)
  g = lax.dot_general(b1h, t1h, (((0,), (0,)), ((), ())),
                      preferred_element_type=jnp.float32)
  part = jnp.dot(g, emb_ref[...], preferred_element_type=jnp.float32)
  cpart = jnp.sum(b1h, axis=0)[:, None]

  @pl.when(i == 0)
  def _():
    tsum_ref[...] = jnp.zeros_like(tsum_ref)
    cnt_ref[...] = jnp.zeros_like(cnt_ref)

  tsum_ref[...] += part
  cnt_ref[...] += jnp.broadcast_to(cpart, cnt_ref.shape)


def _pool_temb(threeDif, batchf, emb):
  return pl.pallas_call(
      _pool_temb_body,
      grid=(_NGRID,),
      in_specs=[
          pl.BlockSpec((_NBLK, 1), lambda i: (i, 0)),
          pl.BlockSpec((_NBLK, 1), lambda i: (i, 0)),
          pl.BlockSpec((21, 128), lambda i: (0, 0)),
      ],
      out_specs=[
          pl.BlockSpec((B, 128), lambda i: (0, 0)),
          pl.BlockSpec((B, 128), lambda i: (0, 0)),
      ],
      out_shape=[jax.ShapeDtypeStruct((B, 128), jnp.float32),
                 jax.ShapeDtypeStruct((B, 128), jnp.float32)],
  )(threeDif, batchf, emb)


def _pool_feats_body(x_ref, bf_ref, out_ref):
  i = pl.program_id(0)
  onehot = (bf_ref[...] == lax.broadcasted_iota(
      jnp.int32, (bf_ref.shape[0], B), 1).astype(jnp.float32)
            ).astype(jnp.float32)
  part = lax.dot_general(onehot, x_ref[...], (((0,), (0,)), ((), ())),
                         preferred_element_type=jnp.float32)

  @pl.when(i == 0)
  def _():
    out_ref[...] = jnp.zeros_like(out_ref)

  out_ref[...] += part


def _pool_feats(xin64, batchf):
  return pl.pallas_call(
      _pool_feats_body,
      grid=(_NGRID,),
      in_specs=[
          pl.BlockSpec((_NBLK, D64), lambda i: (i, 0)),
          pl.BlockSpec((_NBLK, 1), lambda i: (i, 0)),
      ],
      out_specs=pl.BlockSpec((B, D64), lambda i: (0, 0)),
      out_shape=jax.ShapeDtypeStruct((B, D64), jnp.float32),
  )(xin64, batchf)


def _classifier_body(fsum_ref, psum_ref, tsum_ref, cnt_ref,
                     w1x_ref, w1e_ref, w1t_ref, b1_ref, w2_ref, b2_ref,
                     cls_ref, xmean_ref):
  inv = 1.0 / jnp.maximum(cnt_ref[:, 0:1], 1.0)
  x_mean = fsum_ref[:, 3:3 + F] * inv
  esm = psum_ref[...] * inv
  tm = tsum_ref[:, 0:128] * inv
  h1 = jax.nn.relu(
      jnp.dot(x_mean, w1x_ref[...], preferred_element_type=jnp.float32)
      + jnp.dot(esm, w1e_ref[...], preferred_element_type=jnp.float32)
      + jnp.dot(tm, w1t_ref[...], preferred_element_type=jnp.float32)
      + b1_ref[...])
  cls_ref[...] = jnp.dot(h1, w2_ref[...],
                         preferred_element_type=jnp.float32) + b2_ref[...]
  xmean_ref[...] = x_mean


def _classifier(fsum, psum, tsum, cnt, params):
  w1 = params["cls_w1"]
  args = (fsum, psum, tsum, cnt,
          w1[0:F], w1[F:F + 2560], w1[F + 2560:],
          params["cls_b1"].reshape(1, -1),
          params["cls_w2"], params["cls_b2"].reshape(1, -1))
  full = lambda a: pl.BlockSpec(a.shape, lambda: tuple(0 for _ in a.shape))
  return pl.pallas_call(
      _classifier_body,
      in_specs=[full(a) for a in args],
      out_specs=[full(jnp.zeros((B, 384))), full(jnp.zeros((B, F)))],
      out_shape=[jax.ShapeDtypeStruct((B, 384), jnp.float32),
                 jax.ShapeDtypeStruct((B, F), jnp.float32)],
  )(*args)


def kernel(x, pos, mu_r_norm, edge_attr, protein_x, edge_index, batch,
           threeDi_idx, params):
  batchf = batch.astype(jnp.float32).reshape(N, 1)
  threeDif = threeDi_idx.astype(jnp.float32).reshape(N, 1)
  row2d = edge_index[0].reshape(ROWS2D, G)
  col2d = edge_index[1].reshape(ROWS2D, G)
  ea_grp = edge_attr.reshape(E // _EBLK, _E2BLK, 2, 16).transpose(
      0, 2, 1, 3).reshape(E, 16)
  zeros64 = jnp.zeros((N, D64), jnp.float32)

  ne_w = params["ne_w"]
  xin = _prologue(x, mu_r_norm, pos, ne_w[0:128], ne_w[128:],
                  params["ne_b"].reshape(1, -1))
  psum = _pool_protein(protein_x, batchf)
  tsum, cnt = _pool_temb(threeDif, batchf, params["emb"])

  H = 4
  hrows = ROWS2D // H
  hedges = E // H
  sels = _pack_selectors()
  for lp in params["layers"]:
    accs = []
    for h in range(H):
      fr, fc = _sc_gather(xin, row2d, col2d, h * hrows, hrows)
      m2 = _edge_mlp(fr.reshape(hedges // 2, 128),
                     fc.reshape(hedges // 2, 128),
                     ea_grp, lp, h * (hedges // _EBLK), sels)
      accs.append(_sc_scatter(m2.reshape(hedges, D64), row2d, zeros64,
                              h * hrows, hrows))
    xin = _node_update(xin, accs, lp)

  fsum = _pool_feats(xin, batchf)
  cls, x_mean = _classifier(fsum, psum, tsum, cnt, params)
  return cls, x_mean


# speedup vs baseline: 1.0841x; 1.0841x over previous
import jax
import jax.numpy as jnp
from jax import lax
from jax.experimental import pallas as pl
from jax.experimental.pallas import tpu as pltpu
from jax.experimental.pallas import tpu_sc as plsc

N = 10000
E = 640000
B = 32
D64 = 64
F = 32
NC = 2
NS = 16
NW = NC * NS
G = 125
JPC = 4
CHUNK = G * JPC
ROWS2D = E // G
TILE_ROWS = ROWS2D // NW
NCHUNK = TILE_ROWS // JPC
NPT = N // NS
EP2 = E // 2

def _sc_mesh():
  return plsc.VectorSubcoreMesh(
      core_axis_name="c", subcore_axis_name="s", num_cores=NC, num_subcores=NS)


def _sc_gather(xin64, row2d, col2d, hoff, nrows):
  tile_rows = nrows // NW
  nchunk = tile_rows // JPC
  ne = nrows * G

  def body(xin_hbm, row2d_hbm, col2d_hbm, fr_hbm, fc_hbm,
           idx_r, idx_c, buf_r, buf_c, sem):
    c = lax.axis_index("c")
    s = lax.axis_index("s")
    w = s * NC + c
    base2d = hoff + w * tile_rows

    @pl.loop(0, nchunk)
    def _chunk(k):
      rbase = base2d + k * JPC
      pltpu.sync_copy(row2d_hbm.at[pl.ds(rbase, JPC)], idx_r)
      pltpu.sync_copy(col2d_hbm.at[pl.ds(rbase, JPC)], idx_c)
      descs = []
      for j in range(JPC):
        descs.append(pltpu.async_copy(
            xin_hbm.at[idx_r.at[j]], buf_r.at[pl.ds(j * G, G)], sem))
        descs.append(pltpu.async_copy(
            xin_hbm.at[idx_c.at[j]], buf_c.at[pl.ds(j * G, G)], sem))
      for d in descs:
        d.wait()
      ebase = (rbase - hoff) * G
      pltpu.sync_copy(buf_r, fr_hbm.at[pl.ds(ebase, CHUNK)])
      pltpu.sync_copy(buf_c, fc_hbm.at[pl.ds(ebase, CHUNK)])

  return pl.kernel(
      body,
      out_type=(jax.ShapeDtypeStruct((ne, D64), jnp.float32),
                jax.ShapeDtypeStruct((ne, D64), jnp.float32)),
      mesh=_sc_mesh(),
      scratch_types=(
          pltpu.VMEM((JPC, G), jnp.int32),
          pltpu.VMEM((JPC, G), jnp.int32),
          pltpu.VMEM((CHUNK, D64), jnp.float32),
          pltpu.VMEM((CHUNK, D64), jnp.float32),
          pltpu.SemaphoreType.DMA,
      ),
      compiler_params=pltpu.CompilerParams(use_tc_tiling_on_sc=False),
  )(xin64, row2d, col2d)


def _sc_scatter(m64, row2d, zeros64, hoff, nrows):
  tile_rows = nrows // NW
  nchunk = tile_rows // JPC

  def body(m64_hbm, row2d_hbm, zeros_hbm, out_hbm, idx_r, vals, sem, acc):
    c = lax.axis_index("c")
    s = lax.axis_index("s")
    w = s * NC + c
    base2d = hoff + w * tile_rows

    pltpu.sync_copy(zeros_hbm.at[pl.ds(s * NPT, NPT)],
                    acc.at[pl.ds(s * NPT, NPT)])
    plsc.subcore_barrier()

    @pl.loop(0, nchunk)
    def _chunk(k):
      rbase = base2d + k * JPC
      pltpu.sync_copy(row2d_hbm.at[pl.ds(rbase, JPC)], idx_r)
      pltpu.sync_copy(m64_hbm.at[pl.ds((rbase - hoff) * G, CHUNK)], vals)
      descs = []
      for j in range(JPC):
        descs.append(pltpu.async_copy(
            vals.at[pl.ds(j * G, G)], acc.at[idx_r.at[j]], sem, add=True))
      for d in descs:
        d.wait()

    plsc.subcore_barrier()
    pltpu.sync_copy(acc.at[pl.ds(s * NPT, NPT)],
                    out_hbm.at[c, pl.ds(s * NPT, NPT)])

  return pl.kernel(
      body,
      out_type=jax.ShapeDtypeStruct((NC, N, D64), jnp.float32),
      mesh=_sc_mesh(),
      scratch_types=(
          pltpu.VMEM((JPC, G), jnp.int32),
          pltpu.VMEM((CHUNK, D64), jnp.float32),
          pltpu.SemaphoreType.DMA,
          pltpu.VMEM_SHARED((N, D64), jnp.float32),
      ),
      compiler_params=pltpu.CompilerParams(use_tc_tiling_on_sc=False),
  )(m64, row2d, zeros64)


_NBLK = 1000
_NGRID = N // _NBLK
_EBLK = 4000
_E2BLK = _EBLK // 2
_EGRID = E // _EBLK


def _prologue_body(x_ref, mu_ref, pos_ref, wx_ref, wmu_ref, b_ref, out_ref):
  proj = (jnp.dot(x_ref[...], wx_ref[...], preferred_element_type=jnp.float32)
          + jnp.dot(mu_ref[...], wmu_ref[...],
                    preferred_element_type=jnp.float32)
          + b_ref[...])
  pad = jnp.zeros((out_ref.shape[0], D64 - 3 - F), jnp.float32)
  out_ref[...] = jnp.concatenate([pos_ref[...], proj, pad], axis=1)


def _prologue(x, mu, pos, wx, wmu, b):
  return pl.pallas_call(
      _prologue_body,
      grid=(_NGRID,),
      in_specs=[
          pl.BlockSpec((_NBLK, 128), lambda i: (i, 0)),
          pl.BlockSpec((_NBLK, 5), lambda i: (i, 0)),
          pl.BlockSpec((_NBLK, 3), lambda i: (i, 0)),
          pl.BlockSpec((128, F), lambda i: (0, 0)),
          pl.BlockSpec((5, F), lambda i: (0, 0)),
          pl.BlockSpec((1, F), lambda i: (0, 0)),
      ],
      out_specs=pl.BlockSpec((_NBLK, D64), lambda i: (i, 0)),
      out_shape=jax.ShapeDtypeStruct((N, D64), jnp.float32),
  )(x, mu, pos, wx, wmu, b)


def _edge_mlp_body(fr2_ref, fc2_ref, ea_ref,
                   w1r_ref, w1c_ref, w1e_ref, w1d_ref, b1_ref,
                   w2_ref, b2_ref, cw1_ref, cb1_ref, cw2_ref, cb2_ref,
                   sr0_ref, sr1_ref, onesrow_ref,
                   out_ref):
  fr2 = fr2_ref[...]
  fc2 = fc2_ref[...]
  fr = jnp.concatenate([fr2[:, 0:D64], fr2[:, D64:2 * D64]], axis=0)
  fc = jnp.concatenate([fc2[:, 0:D64], fc2[:, D64:2 * D64]], axis=0)
  ea = ea_ref[...]
  rel = fr[:, 0:3] - fc[:, 0:3]
  rd = jnp.sum(rel * rel, axis=1, keepdims=True)
  hpre = (jnp.dot(fr[:, 3:3 + F], w1r_ref[...],
                  preferred_element_type=jnp.float32)
          + jnp.dot(fc[:, 3:3 + F], w1c_ref[...],
                    preferred_element_type=jnp.float32)
          + jnp.dot(ea, w1e_ref[...],
                    preferred_element_type=jnp.float32)
          + rd * w1d_ref[...] + b1_ref[...])
  h = jax.nn.silu(hpre)
  m = jax.nn.silu(jnp.dot(h, w2_ref[...],
                          preferred_element_type=jnp.float32) + b2_ref[...])
  h2 = jax.nn.silu(jnp.dot(m, cw1_ref[...],
                           preferred_element_type=jnp.float32) + cb1_ref[...])
  cw = jnp.dot(h2, cw2_ref[...],
               preferred_element_type=jnp.float32) + cb2_ref[...]
  rc = rel * cw
  zpad = jnp.zeros((_EBLK, D64 - F), jnp.float32)
  m64 = jnp.concatenate([m, zpad], axis=1)
  out_ref[...] = (
      jnp.concatenate([m64[0:_E2BLK], m64[_E2BLK:_EBLK]], axis=1)
      + jnp.dot(rc[0:_E2BLK], sr0_ref[...], preferred_element_type=jnp.float32)
      + jnp.dot(rc[_E2BLK:], sr1_ref[...], preferred_element_type=jnp.float32)
      + onesrow_ref[...])


def _pack_selectors():
  i32 = jnp.int32
  r = lambda n: jnp.arange(n)
  sr0 = jnp.zeros((3, 128), jnp.float32).at[r(3), F + r(3)].set(1.0)
  sr1 = jnp.zeros((3, 128), jnp.float32).at[r(3), D64 + F + r(3)].set(1.0)
  onesrow = jnp.zeros((1, 128), jnp.float32).at[
      0, jnp.array([F + 3, D64 + F + 3], i32)].set(1.0)
  return sr0, sr1, onesrow


def _edge_mlp(fr2, fc2, edge_attr, lp, hblk, sels):
  ein = 2 * F + 16 + 1
  w1 = lp["e_w1"]
  args = (fr2, fc2, edge_attr,
          w1[0:F], w1[F:2 * F], w1[2 * F:2 * F + 16], w1[2 * F + 16:ein],
          lp["e_b1"].reshape(1, -1),
          lp["e_w2"], lp["e_b2"].reshape(1, -1),
          lp["c_w1"], lp["c_b1"].reshape(1, -1),
          lp["c_w2"], lp["c_b2"].reshape(1, -1)) + sels
  full = lambda a: pl.BlockSpec(a.shape, lambda i: tuple(0 for _ in a.shape))
  return pl.pallas_call(
      _edge_mlp_body,
      grid=(fr2.shape[0] // _E2BLK,),
      in_specs=[
          pl.BlockSpec((_E2BLK, 128), lambda i: (i, 0)),
          pl.BlockSpec((_E2BLK, 128), lambda i: (i, 0)),
          pl.BlockSpec((_EBLK, 16), lambda i: (i + hblk, 0)),
      ] + [full(a) for a in args[3:]],
      out_specs=pl.BlockSpec((_E2BLK, 128), lambda i: (i, 0)),
      out_shape=jax.ShapeDtypeStruct((fr2.shape[0], 128), jnp.float32),
  )(*args)


def _node_update_body(n_acc, *refs):
  (xin_ref, *rest) = refs
  acc_refs = rest[:n_acc]
  w1f_ref, w1m_ref, b1_ref, w2_ref, b2_ref, out_ref = rest[n_acc:]
  xin = xin_ref[...]
  a = acc_refs[0][...]
  for r in acc_refs[1:]:
    a = a + r[...]
  m_i = a[:, 0:F]
  csum = a[:, F:F + 3]
  deg = a[:, F + 3:F + 4]
  coors = xin[:, 0:3]
  feats = xin[:, 3:3 + F]
  coors_out = coors + csum / jnp.maximum(deg, 1.0)
  nh = jax.nn.silu(
      jnp.dot(feats, w1f_ref[...], preferred_element_type=jnp.float32)
      + jnp.dot(m_i, w1m_ref[...], preferred_element_type=jnp.float32)
      + b1_ref[...])
  feats_out = feats + jnp.dot(nh, w2_ref[...],
                              preferred_element_type=jnp.float32) + b2_ref[...]
  pad = jnp.zeros((out_ref.shape[0], D64 - 3 - F), jnp.float32)
  out_ref[...] = jnp.concatenate(
      [coors + coors_out, feats + feats_out, pad], axis=1)


def _node_update(xin64, accs, lp):
  import functools
  w1 = lp["n_w1"]
  acc_args = tuple(a[i] for a in accs for i in range(NC))
  n_acc = len(acc_args)
  args = (xin64,) + acc_args + (w1[0:F], w1[F:2 * F],
          lp["n_b1"].reshape(1, -1), lp["n_w2"], lp["n_b2"].reshape(1, -1))
  full = lambda a: pl.BlockSpec(a.shape, lambda i: tuple(0 for _ in a.shape))
  nspec = pl.BlockSpec((_NBLK, D64), lambda i: (i, 0))
  return pl.pallas_call(
      functools.partial(_node_update_body, n_acc),
      grid=(_NGRID,),
      in_specs=[nspec] * (1 + n_acc) + [full(a) for a in args[1 + n_acc:]],
      out_specs=pl.BlockSpec((_NBLK, D64), lambda i: (i, 0)),
      out_shape=jax.ShapeDtypeStruct((N, D64), jnp.float32),
  )(*args)


def _pool_protein_body(p_ref, bf_ref, out_ref):
  i = pl.program_id(1)
  onehot = (bf_ref[...] == lax.broadcasted_iota(
      jnp.int32, (bf_ref.shape[0], B), 1).astype(jnp.float32)
            ).astype(jnp.float32)
  part = lax.dot_general(onehot, p_ref[...], (((0,), (0,)), ((), ())),
                         preferred_element_type=jnp.float32)

  @pl.when(i == 0)
  def _():
    out_ref[...] = jnp.zeros_like(out_ref)

  out_ref[...] += part


def _pool_protein(protein_x, batchf):
  FB = 512
  return pl.pallas_call(
      _pool_protein_body,
      grid=(2560 // FB, _NGRID),
      in_specs=[
          pl.BlockSpec((_NBLK, FB), lambda j, i: (i, j)),
          pl.BlockSpec((_NBLK, 1), lambda j, i: (i, 0)),
      ],
      out_specs=pl.BlockSpec((B, FB), lambda j, i: (0, j)),
      out_shape=jax.ShapeDtypeStruct((B, 2560), jnp.float32),
  )(protein_x, batchf)


def _pool_temb_body(t_ref, bf_ref, emb_ref, tsum_ref, cnt_ref):
  i = pl.program_id(0)
  rows = t_ref.shape[0]
  b1h = (bf_ref[...] == lax.broadcasted_iota(
      jnp.int32, (rows, B), 1).astype(jnp.float32)).astype(jnp.float32)
  t1h = (t_ref[...] == lax.broadcasted_iota(
      jnp.int32, (rows, 21), 1).astype(jnp.float32)).astype(jnp.float32)
  g = lax.dot_general(b1h, t1h, (((0,), (0,)), ((), ())),
                      preferred_element_type=jnp.float32)
  part = jnp.dot(g, emb_ref[...], preferred_element_type=jnp.float32)
  cpart = jnp.sum(b1h, axis=0)[:, None]

  @pl.when(i == 0)
  def _():
    tsum_ref[...] = jnp.zeros_like(tsum_ref)
    cnt_ref[...] = jnp.zeros_like(cnt_ref)

  tsum_ref[...] += part
  cnt_ref[...] += jnp.broadcast_to(cpart, cnt_ref.shape)


def _pool_temb(threeDif, batchf, emb):
  return pl.pallas_call(
      _pool_temb_body,
      grid=(_NGRID,),
      in_specs=[
          pl.BlockSpec((_NBLK, 1), lambda i: (i, 0)),
          pl.BlockSpec((_NBLK, 1), lambda i: (i, 0)),
          pl.BlockSpec((21, 128), lambda i: (0, 0)),
      ],
      out_specs=[
          pl.BlockSpec((B, 128), lambda i: (0, 0)),
          pl.BlockSpec((B, 128), lambda i: (0, 0)),
      ],
      out_shape=[jax.ShapeDtypeStruct((B, 128), jnp.float32),
                 jax.ShapeDtypeStruct((B, 128), jnp.float32)],
  )(threeDif, batchf, emb)


def _pool_feats_body(x_ref, bf_ref, out_ref):
  i = pl.program_id(0)
  onehot = (bf_ref[...] == lax.broadcasted_iota(
      jnp.int32, (bf_ref.shape[0], B), 1).astype(jnp.float32)
            ).astype(jnp.float32)
  part = lax.dot_general(onehot, x_ref[...], (((0,), (0,)), ((), ())),
                         preferred_element_type=jnp.float32)

  @pl.when(i == 0)
  def _():
    out_ref[...] = jnp.zeros_like(out_ref)

  out_ref[...] += part


def _pool_feats(xin64, batchf):
  return pl.pallas_call(
      _pool_feats_body,
      grid=(_NGRID,),
      in_specs=[
          pl.BlockSpec((_NBLK, D64), lambda i: (i, 0)),
          pl.BlockSpec((_NBLK, 1), lambda i: (i, 0)),
      ],
      out_specs=pl.BlockSpec((B, D64), lambda i: (0, 0)),
      out_shape=jax.ShapeDtypeStruct((B, D64), jnp.float32),
  )(xin64, batchf)


def _classifier_body(fsum_ref, psum_ref, tsum_ref, cnt_ref,
                     w1x_ref, w1e_ref, w1t_ref, b1_ref, w2_ref, b2_ref,
                     cls_ref, xmean_ref):
  inv = 1.0 / jnp.maximum(cnt_ref[:, 0:1], 1.0)
  x_mean = fsum_ref[:, 3:3 + F] * inv
  esm = psum_ref[...] * inv
  tm = tsum_ref[:, 0:128] * inv
  h1 = jax.nn.relu(
      jnp.dot(x_mean, w1x_ref[...], preferred_element_type=jnp.float32)
      + jnp.dot(esm, w1e_ref[...], preferred_element_type=jnp.float32)
      + jnp.dot(tm, w1t_ref[...], preferred_element_type=jnp.float32)
      + b1_ref[...])
  cls_ref[...] = jnp.dot(h1, w2_ref[...],
                         preferred_element_type=jnp.float32) + b2_ref[...]
  xmean_ref[...] = x_mean


def _classifier(fsum, psum, tsum, cnt, params):
  w1 = params["cls_w1"]
  args = (fsum, psum, tsum, cnt,
          w1[0:F], w1[F:F + 2560], w1[F + 2560:],
          params["cls_b1"].reshape(1, -1),
          params["cls_w2"], params["cls_b2"].reshape(1, -1))
  full = lambda a: pl.BlockSpec(a.shape, lambda: tuple(0 for _ in a.shape))
  return pl.pallas_call(
      _classifier_body,
      in_specs=[full(a) for a in args],
      out_specs=[full(jnp.zeros((B, 384))), full(jnp.zeros((B, F)))],
      out_shape=[jax.ShapeDtypeStruct((B, 384), jnp.float32),
                 jax.ShapeDtypeStruct((B, F), jnp.float32)],
  )(*args)


def kernel(x, pos, mu_r_norm, edge_attr, protein_x, edge_index, batch,
           threeDi_idx, params):
  batchf = batch.astype(jnp.float32).reshape(N, 1)
  threeDif = threeDi_idx.astype(jnp.float32).reshape(N, 1)
  row2d = edge_index[0].reshape(ROWS2D, G)
  col2d = edge_index[1].reshape(ROWS2D, G)
  ea_grp = edge_attr.reshape(E // _EBLK, _E2BLK, 2, 16).transpose(
      0, 2, 1, 3).reshape(E, 16)
  zeros64 = jnp.zeros((N, D64), jnp.float32)

  ne_w = params["ne_w"]
  xin = _prologue(x, mu_r_norm, pos, ne_w[0:128], ne_w[128:],
                  params["ne_b"].reshape(1, -1))
  psum = _pool_protein(protein_x, batchf)
  tsum, cnt = _pool_temb(threeDif, batchf, params["emb"])

  H = 4
  hrows = ROWS2D // H
  hedges = E // H
  sels = _pack_selectors()
  for lp in params["layers"]:
    accs = []
    for h in range(H):
      fr, fc = _sc_gather(xin, row2d, col2d, h * hrows, hrows)
      m2 = _edge_mlp(fr.reshape(hedges // 2, 128),
                     fc.reshape(hedges // 2, 128),
                     ea_grp, lp, h * (hedges // _EBLK), sels)
      accs.append(_sc_scatter(m2.reshape(hedges, D64), row2d, zeros64,
                              h * hrows, hrows))
    xin = _node_update(xin, accs, lp)

  fsum = _pool_feats(xin, batchf)
  cls, x_mean = _classifier(fsum, psum, tsum, cnt, params)
  return cls, x_mean


# speedup vs baseline: 1.1485x; 1.0594x over previous
import jax
import jax.numpy as jnp
from jax import lax
from jax.experimental import pallas as pl
from jax.experimental.pallas import tpu as pltpu
from jax.experimental.pallas import tpu_sc as plsc

N = 10000
E = 640000
B = 32
D64 = 64
F = 32
NC = 2
NS = 16
NW = NC * NS
G = 125
JPC = 4
CHUNK = G * JPC
ROWS2D = E // G
TILE_ROWS = ROWS2D // NW
NCHUNK = TILE_ROWS // JPC
NPT = N // NS
EP2 = E // 2

def _sc_mesh():
  return plsc.VectorSubcoreMesh(
      core_axis_name="c", subcore_axis_name="s", num_cores=NC, num_subcores=NS)


def _sc_gather(xin64, row2d, col2d, hoff, nrows):
  tile_rows = nrows // NW
  nchunk = tile_rows // JPC
  ne = nrows * G

  def body(xin_hbm, row2d_hbm, col2d_hbm, fr_hbm, fc_hbm,
           idx_r, idx_c, buf_r, buf_c, sem):
    c = lax.axis_index("c")
    s = lax.axis_index("s")
    w = s * NC + c
    base2d = hoff + w * tile_rows

    @pl.loop(0, nchunk)
    def _chunk(k):
      rbase = base2d + k * JPC
      pltpu.sync_copy(row2d_hbm.at[pl.ds(rbase, JPC)], idx_r)
      pltpu.sync_copy(col2d_hbm.at[pl.ds(rbase, JPC)], idx_c)
      descs = []
      for j in range(JPC):
        descs.append(pltpu.async_copy(
            xin_hbm.at[idx_r.at[j]], buf_r.at[pl.ds(j * G, G)], sem))
        descs.append(pltpu.async_copy(
            xin_hbm.at[idx_c.at[j]], buf_c.at[pl.ds(j * G, G)], sem))
      for d in descs:
        d.wait()
      ebase = (rbase - hoff) * G
      pltpu.sync_copy(buf_r, fr_hbm.at[pl.ds(ebase, CHUNK)])
      pltpu.sync_copy(buf_c, fc_hbm.at[pl.ds(ebase, CHUNK)])

  return pl.kernel(
      body,
      out_type=(jax.ShapeDtypeStruct((ne, D64), jnp.float32),
                jax.ShapeDtypeStruct((ne, D64), jnp.float32)),
      mesh=_sc_mesh(),
      scratch_types=(
          pltpu.VMEM((JPC, G), jnp.int32),
          pltpu.VMEM((JPC, G), jnp.int32),
          pltpu.VMEM((CHUNK, D64), jnp.float32),
          pltpu.VMEM((CHUNK, D64), jnp.float32),
          pltpu.SemaphoreType.DMA,
      ),
      compiler_params=pltpu.CompilerParams(use_tc_tiling_on_sc=False),
  )(xin64, row2d, col2d)


def _sc_scatter(m64, row2d, zeros64, hoff, nrows):
  tile_rows = nrows // NW
  nchunk = tile_rows // JPC

  def body(m64_hbm, row2d_hbm, zeros_hbm, out_hbm, idx_r, vals, sem, acc):
    c = lax.axis_index("c")
    s = lax.axis_index("s")
    w = s * NC + c
    base2d = hoff + w * tile_rows

    pltpu.sync_copy(zeros_hbm.at[pl.ds(s * NPT, NPT)],
                    acc.at[pl.ds(s * NPT, NPT)])
    plsc.subcore_barrier()

    @pl.loop(0, nchunk)
    def _chunk(k):
      rbase = base2d + k * JPC
      pltpu.sync_copy(row2d_hbm.at[pl.ds(rbase, JPC)], idx_r)
      pltpu.sync_copy(m64_hbm.at[pl.ds((rbase - hoff) * G, CHUNK)], vals)
      descs = []
      for j in range(JPC):
        descs.append(pltpu.async_copy(
            vals.at[pl.ds(j * G, G)], acc.at[idx_r.at[j]], sem, add=True))
      for d in descs:
        d.wait()

    plsc.subcore_barrier()
    pltpu.sync_copy(acc.at[pl.ds(s * NPT, NPT)],
                    out_hbm.at[c, pl.ds(s * NPT, NPT)])

  return pl.kernel(
      body,
      out_type=jax.ShapeDtypeStruct((NC, N, D64), jnp.float32),
      mesh=_sc_mesh(),
      scratch_types=(
          pltpu.VMEM((JPC, G), jnp.int32),
          pltpu.VMEM((CHUNK, D64), jnp.float32),
          pltpu.SemaphoreType.DMA,
          pltpu.VMEM_SHARED((N, D64), jnp.float32),
      ),
      compiler_params=pltpu.CompilerParams(use_tc_tiling_on_sc=False),
  )(m64, row2d, zeros64)


_NBLK = 1000
_NGRID = N // _NBLK
_EBLK = 8000
_E2BLK = _EBLK // 2
_EGRID = E // _EBLK


def _prologue_body(x_ref, mu_ref, pos_ref, wx_ref, wmu_ref, b_ref, out_ref):
  proj = (jnp.dot(x_ref[...], wx_ref[...], preferred_element_type=jnp.float32)
          + jnp.dot(mu_ref[...], wmu_ref[...],
                    preferred_element_type=jnp.float32)
          + b_ref[...])
  pad = jnp.zeros((out_ref.shape[0], D64 - 3 - F), jnp.float32)
  out_ref[...] = jnp.concatenate([pos_ref[...], proj, pad], axis=1)


def _prologue(x, mu, pos, wx, wmu, b):
  return pl.pallas_call(
      _prologue_body,
      grid=(_NGRID,),
      in_specs=[
          pl.BlockSpec((_NBLK, 128), lambda i: (i, 0)),
          pl.BlockSpec((_NBLK, 5), lambda i: (i, 0)),
          pl.BlockSpec((_NBLK, 3), lambda i: (i, 0)),
          pl.BlockSpec((128, F), lambda i: (0, 0)),
          pl.BlockSpec((5, F), lambda i: (0, 0)),
          pl.BlockSpec((1, F), lambda i: (0, 0)),
      ],
      out_specs=pl.BlockSpec((_NBLK, D64), lambda i: (i, 0)),
      out_shape=jax.ShapeDtypeStruct((N, D64), jnp.float32),
  )(x, mu, pos, wx, wmu, b)


def _edge_mlp_body(fr2_ref, fc2_ref, ea_ref,
                   w1r_ref, w1c_ref, w1e_ref, w1d_ref, b1_ref,
                   w2_ref, b2_ref, cw1_ref, cb1_ref, cw2_ref, cb2_ref,
                   out_ref):
  fr2 = fr2_ref[...]
  fc2 = fc2_ref[...]
  fr = jnp.concatenate([fr2[:, 0:D64], fr2[:, D64:2 * D64]], axis=0)
  fc = jnp.concatenate([fc2[:, 0:D64], fc2[:, D64:2 * D64]], axis=0)
  ea = ea_ref[...]
  rel = fr[:, 0:3] - fc[:, 0:3]
  rd = jnp.sum(rel * rel, axis=1, keepdims=True)
  hpre = (jnp.dot(fr[:, 3:3 + F], w1r_ref[...],
                  preferred_element_type=jnp.float32)
          + jnp.dot(fc[:, 3:3 + F], w1c_ref[...],
                    preferred_element_type=jnp.float32)
          + jnp.dot(ea, w1e_ref[...],
                    preferred_element_type=jnp.float32)
          + rd * w1d_ref[...] + b1_ref[...])
  h = jax.nn.silu(hpre)
  m = jax.nn.silu(jnp.dot(h, w2_ref[...],
                          preferred_element_type=jnp.float32) + b2_ref[...])
  h2 = jax.nn.silu(jnp.dot(m, cw1_ref[...],
                           preferred_element_type=jnp.float32) + cb1_ref[...])
  cw = jnp.dot(h2, cw2_ref[...],
               preferred_element_type=jnp.float32) + cb2_ref[...]
  ones = jnp.ones((_EBLK, 1), jnp.float32)
  pad = jnp.zeros((_EBLK, D64 - F - 4), jnp.float32)
  payload = jnp.concatenate([m, rel * cw, ones, pad], axis=1)
  out_ref[...] = jnp.concatenate(
      [payload[0:_E2BLK], payload[_E2BLK:_EBLK]], axis=1)


def _edge_mlp(fr2, fc2, edge_attr, lp, hblk):
  ein = 2 * F + 16 + 1
  w1 = lp["e_w1"]
  args = (fr2, fc2, edge_attr,
          w1[0:F], w1[F:2 * F], w1[2 * F:2 * F + 16], w1[2 * F + 16:ein],
          lp["e_b1"].reshape(1, -1),
          lp["e_w2"], lp["e_b2"].reshape(1, -1),
          lp["c_w1"], lp["c_b1"].reshape(1, -1),
          lp["c_w2"], lp["c_b2"].reshape(1, -1))
  full = lambda a: pl.BlockSpec(a.shape, lambda i: tuple(0 for _ in a.shape))
  return pl.pallas_call(
      _edge_mlp_body,
      grid=(fr2.shape[0] // _E2BLK,),
      in_specs=[
          pl.BlockSpec((_E2BLK, 128), lambda i: (i, 0)),
          pl.BlockSpec((_E2BLK, 128), lambda i: (i, 0)),
          pl.BlockSpec((_EBLK, 16), lambda i: (i + hblk, 0)),
      ] + [full(a) for a in args[3:]],
      out_specs=pl.BlockSpec((_E2BLK, 128), lambda i: (i, 0)),
      out_shape=jax.ShapeDtypeStruct((fr2.shape[0], 128), jnp.float32),
  )(*args)


def _node_update_body(n_acc, *refs):
  (xin_ref, *rest) = refs
  acc_refs = rest[:n_acc]
  w1f_ref, w1m_ref, b1_ref, w2_ref, b2_ref, out_ref = rest[n_acc:]
  xin = xin_ref[...]
  a = acc_refs[0][...]
  for r in acc_refs[1:]:
    a = a + r[...]
  m_i = a[:, 0:F]
  csum = a[:, F:F + 3]
  deg = a[:, F + 3:F + 4]
  coors = xin[:, 0:3]
  feats = xin[:, 3:3 + F]
  coors_out = coors + csum / jnp.maximum(deg, 1.0)
  nh = jax.nn.silu(
      jnp.dot(feats, w1f_ref[...], preferred_element_type=jnp.float32)
      + jnp.dot(m_i, w1m_ref[...], preferred_element_type=jnp.float32)
      + b1_ref[...])
  feats_out = feats + jnp.dot(nh, w2_ref[...],
                              preferred_element_type=jnp.float32) + b2_ref[...]
  pad = jnp.zeros((out_ref.shape[0], D64 - 3 - F), jnp.float32)
  out_ref[...] = jnp.concatenate(
      [coors + coors_out, feats + feats_out, pad], axis=1)


def _node_update(xin64, accs, lp):
  import functools
  w1 = lp["n_w1"]
  acc_args = tuple(a[i] for a in accs for i in range(NC))
  n_acc = len(acc_args)
  args = (xin64,) + acc_args + (w1[0:F], w1[F:2 * F],
          lp["n_b1"].reshape(1, -1), lp["n_w2"], lp["n_b2"].reshape(1, -1))
  full = lambda a: pl.BlockSpec(a.shape, lambda i: tuple(0 for _ in a.shape))
  nspec = pl.BlockSpec((_NBLK, D64), lambda i: (i, 0))
  return pl.pallas_call(
      functools.partial(_node_update_body, n_acc),
      grid=(_NGRID,),
      in_specs=[nspec] * (1 + n_acc) + [full(a) for a in args[1 + n_acc:]],
      out_specs=pl.BlockSpec((_NBLK, D64), lambda i: (i, 0)),
      out_shape=jax.ShapeDtypeStruct((N, D64), jnp.float32),
  )(*args)


def _pool_protein_body(p_ref, bf_ref, out_ref):
  i = pl.program_id(1)
  onehot = (bf_ref[...] == lax.broadcasted_iota(
      jnp.int32, (bf_ref.shape[0], B), 1).astype(jnp.float32)
            ).astype(jnp.float32)
  part = lax.dot_general(onehot, p_ref[...], (((0,), (0,)), ((), ())),
                         preferred_element_type=jnp.float32)

  @pl.when(i == 0)
  def _():
    out_ref[...] = jnp.zeros_like(out_ref)

  out_ref[...] += part


def _pool_protein(protein_x, batchf):
  FB = 512
  return pl.pallas_call(
      _pool_protein_body,
      grid=(2560 // FB, _NGRID),
      in_specs=[
          pl.BlockSpec((_NBLK, FB), lambda j, i: (i, j)),
          pl.BlockSpec((_NBLK, 1), lambda j, i: (i, 0)),
      ],
      out_specs=pl.BlockSpec((B, FB), lambda j, i: (0, j)),
      out_shape=jax.ShapeDtypeStruct((B, 2560), jnp.float32),
  )(protein_x, batchf)


def _pool_temb_body(t_ref, bf_ref, emb_ref, tsum_ref, cnt_ref):
  i = pl.program_id(0)
  rows = t_ref.shape[0]
  b1h = (bf_ref[...] == lax.broadcasted_iota(
      jnp.int32, (rows, B), 1).astype(jnp.float32)).astype(jnp.float32)
  t1h = (t_ref[...] == lax.broadcasted_iota(
      jnp.int32, (rows, 21), 1).astype(jnp.float32)).astype(jnp.float32)
  g = lax.dot_general(b1h, t1h, (((0,), (0,)), ((), ())),
                      preferred_element_type=jnp.float32)
  part = jnp.dot(g, emb_ref[...], preferred_element_type=jnp.float32)
  cpart = jnp.sum(b1h, axis=0)[:, None]

  @pl.when(i == 0)
  def _():
    tsum_ref[...] = jnp.zeros_like(tsum_ref)
    cnt_ref[...] = jnp.zeros_like(cnt_ref)

  tsum_ref[...] += part
  cnt_ref[...] += jnp.broadcast_to(cpart, cnt_ref.shape)


def _pool_temb(threeDif, batchf, emb):
  return pl.pallas_call(
      _pool_temb_body,
      grid=(_NGRID,),
      in_specs=[
          pl.BlockSpec((_NBLK, 1), lambda i: (i, 0)),
          pl.BlockSpec((_NBLK, 1), lambda i: (i, 0)),
          pl.BlockSpec((21, 128), lambda i: (0, 0)),
      ],
      out_specs=[
          pl.BlockSpec((B, 128), lambda i: (0, 0)),
          pl.BlockSpec((B, 128), lambda i: (0, 0)),
      ],
      out_shape=[jax.ShapeDtypeStruct((B, 128), jnp.float32),
                 jax.ShapeDtypeStruct((B, 128), jnp.float32)],
  )(threeDif, batchf, emb)


def _pool_feats_body(x_ref, bf_ref, out_ref):
  i = pl.program_id(0)
  onehot = (bf_ref[...] == lax.broadcasted_iota(
      jnp.int32, (bf_ref.shape[0], B), 1).astype(jnp.float32)
            ).astype(jnp.float32)
  part = lax.dot_general(onehot, x_ref[...], (((0,), (0,)), ((), ())),
                         preferred_element_type=jnp.float32)

  @pl.when(i == 0)
  def _():
    out_ref[...] = jnp.zeros_like(out_ref)

  out_ref[...] += part


def _pool_feats(xin64, batchf):
  return pl.pallas_call(
      _pool_feats_body,
      grid=(_NGRID,),
      in_specs=[
          pl.BlockSpec((_NBLK, D64), lambda i: (i, 0)),
          pl.BlockSpec((_NBLK, 1), lambda i: (i, 0)),
      ],
      out_specs=pl.BlockSpec((B, D64), lambda i: (0, 0)),
      out_shape=jax.ShapeDtypeStruct((B, D64), jnp.float32),
  )(xin64, batchf)


def _classifier_body(fsum_ref, psum_ref, tsum_ref, cnt_ref,
                     w1x_ref, w1e_ref, w1t_ref, b1_ref, w2_ref, b2_ref,
                     cls_ref, xmean_ref):
  inv = 1.0 / jnp.maximum(cnt_ref[:, 0:1], 1.0)
  x_mean = fsum_ref[:, 3:3 + F] * inv
  esm = psum_ref[...] * inv
  tm = tsum_ref[:, 0:128] * inv
  h1 = jax.nn.relu(
      jnp.dot(x_mean, w1x_ref[...], preferred_element_type=jnp.float32)
      + jnp.dot(esm, w1e_ref[...], preferred_element_type=jnp.float32)
      + jnp.dot(tm, w1t_ref[...], preferred_element_type=jnp.float32)
      + b1_ref[...])
  cls_ref[...] = jnp.dot(h1, w2_ref[...],
                         preferred_element_type=jnp.float32) + b2_ref[...]
  xmean_ref[...] = x_mean


def _classifier(fsum, psum, tsum, cnt, params):
  w1 = params["cls_w1"]
  args = (fsum, psum, tsum, cnt,
          w1[0:F], w1[F:F + 2560], w1[F + 2560:],
          params["cls_b1"].reshape(1, -1),
          params["cls_w2"], params["cls_b2"].reshape(1, -1))
  full = lambda a: pl.BlockSpec(a.shape, lambda: tuple(0 for _ in a.shape))
  return pl.pallas_call(
      _classifier_body,
      in_specs=[full(a) for a in args],
      out_specs=[full(jnp.zeros((B, 384))), full(jnp.zeros((B, F)))],
      out_shape=[jax.ShapeDtypeStruct((B, 384), jnp.float32),
                 jax.ShapeDtypeStruct((B, F), jnp.float32)],
  )(*args)


def kernel(x, pos, mu_r_norm, edge_attr, protein_x, edge_index, batch,
           threeDi_idx, params):
  batchf = batch.astype(jnp.float32).reshape(N, 1)
  threeDif = threeDi_idx.astype(jnp.float32).reshape(N, 1)
  row2d = edge_index[0].reshape(ROWS2D, G)
  col2d = edge_index[1].reshape(ROWS2D, G)
  ea_grp = edge_attr.reshape(E // _EBLK, _E2BLK, 2, 16).transpose(
      0, 2, 1, 3).reshape(E, 16)
  zeros64 = jnp.zeros((N, D64), jnp.float32)

  ne_w = params["ne_w"]
  xin = _prologue(x, mu_r_norm, pos, ne_w[0:128], ne_w[128:],
                  params["ne_b"].reshape(1, -1))
  psum = _pool_protein(protein_x, batchf)
  tsum, cnt = _pool_temb(threeDif, batchf, params["emb"])

  H = 4
  hrows = ROWS2D // H
  hedges = E // H
  for lp in params["layers"]:
    accs = []
    for h in range(H):
      fr, fc = _sc_gather(xin, row2d, col2d, h * hrows, hrows)
      m2 = _edge_mlp(fr.reshape(hedges // 2, 128),
                     fc.reshape(hedges // 2, 128),
                     ea_grp, lp, h * (hedges // _EBLK))
      accs.append(_sc_scatter(m2.reshape(hedges, D64), row2d, zeros64,
                              h * hrows, hrows))
    xin = _node_update(xin, accs, lp)

  fsum = _pool_feats(xin, batchf)
  cls, x_mean = _classifier(fsum, psum, tsum, cnt, params)
  return cls, x_mean
